# baseline (speedup 1.0000x reference)
import functools

import jax
import jax.numpy as jnp
import numpy as np
from jax import lax
from jax.experimental import pallas as pl
from jax.experimental.pallas import tpu as pltpu
from jax.experimental.pallas import tpu_sc as plsc

_T = 2048
_F = _T // 2 + 1
_FPAD = 1152
_K = 4
_PMAX = 64
_PMIN = 32
_NP = _PMAX - _PMIN + 1
_FOLDPAD = 2176


def _build_wdft() -> np.ndarray:
    t = np.arange(_T, dtype=np.float64)
    f = np.arange(_F, dtype=np.float64)
    ang = (2.0 * np.pi / _T) * np.outer(t, f)
    w = np.zeros((_T, 2 * _FPAD), dtype=np.float32)
    w[:, :_F] = np.cos(ang).astype(np.float32)
    w[:, _FPAD:_FPAD + _F] = -np.sin(ang).astype(np.float32)
    w[:, 0] = 0.0
    w[:, _FPAD] = 0.0
    return w


def _build_wfold() -> np.ndarray:
    w = np.zeros((_T, _FOLDPAD), dtype=np.float32)
    for j in range(_NP):
        p = _PMIN + j
        cycles = _T // p
        start = _T - cycles * p
        tt = np.arange(start, _T)
        w[tt, j * _PMAX + ((tt - start) % p)] = 1.0
    return w


_WDFT = _build_wdft()
_WFOLD = _build_wfold().astype(jnp.bfloat16)
_INV_CYCLES = [1.0 / (_T // (_PMIN + j)) for j in range(_NP)]


def _build_lut() -> np.ndarray:
    lut = np.zeros((256,), dtype=np.float32)
    for kk in range(128):
        p = int(np.clip(_T // max(kk, 1), _PMIN, _PMAX))
        lut[kk] = (p - _PMIN) * _PMAX
        lut[128 + kk] = 1.0 / (_T // p)
    return lut


_LUT = _build_lut()


def _dft_body(x_ref, w_ref, y_ref):
    y_ref[...] = jax.lax.dot_general(
        x_ref[...], w_ref[...], (((1,), (0,)), ((), ())),
        preferred_element_type=jnp.float32,
        precision=jax.lax.Precision.HIGHEST,
    )


def _fold_body(xh_ref, xl_ref, w_ref, y_ref):
    dn = (((1,), (0,)), ((), ()))
    y_ref[...] = (
        jax.lax.dot_general(xh_ref[...], w_ref[...], dn,
                            preferred_element_type=jnp.float32)
        + jax.lax.dot_general(xl_ref[...], w_ref[...], dn,
                              preferred_element_type=jnp.float32)
    )


_BN = 1024
_LANES = 16
_NWORK = 32
_ROWS_PER_W = _BN // _NWORK
_NCHUNK = _FPAD // _LANES


def _sc_select(ydft_hbm, yfold_hbm, lut_hbm, out_hbm, spec_v, fold_v, o_v,
               lut_v):
    wid = lax.axis_index("s") * 2 + lax.axis_index("c")
    iota = lax.broadcasted_iota(jnp.int32, (_LANES,), 0)
    neg1 = jnp.full((_LANES,), -1.0, jnp.float32)
    big = jnp.full((_LANES,), 1 << 30, jnp.int32)
    pltpu.sync_copy(lut_hbm, lut_v)

    def row_body(r, _):
        row = wid * _ROWS_PER_W + r
        pltpu.sync_copy(ydft_hbm.at[pl.ds(row * (2 * _FPAD), 2 * _FPAD)],
                        spec_v)
        pltpu.sync_copy(yfold_hbm.at[pl.ds(row * _FOLDPAD, _FOLDPAD)],
                        fold_v)

        def chunk_body(i, carry):
            v1, v2, v3, v4, i1, i2, i3, i4 = carry
            c = spec_v[pl.ds(i * _LANES, _LANES)]
            s = spec_v[pl.ds(_FPAD + i * _LANES, _LANES)]
            m = c * c + s * s
            gi = iota + i * _LANES
            b1 = m > v1
            nv1 = jnp.where(b1, m, v1)
            ni1 = jnp.where(b1, gi, i1)
            m2 = jnp.where(b1, v1, m)
            g2 = jnp.where(b1, i1, gi)
            b2 = m2 > v2
            nv2 = jnp.where(b2, m2, v2)
            ni2 = jnp.where(b2, g2, i2)
            m3 = jnp.where(b2, v2, m2)
            g3 = jnp.where(b2, i2, g2)
            b3 = m3 > v3
            nv3 = jnp.where(b3, m3, v3)
            ni3 = jnp.where(b3, g3, i3)
            m4 = jnp.where(b3, v3, m3)
            g4 = jnp.where(b3, i3, g3)
            b4 = m4 > v4
            nv4 = jnp.where(b4, m4, v4)
            ni4 = jnp.where(b4, g4, i4)
            return nv1, nv2, nv3, nv4, ni1, ni2, ni3, ni4

        init = (neg1, neg1, neg1, neg1, big, big, big, big)
        v1, v2, v3, v4, i1, i2, i3, i4 = lax.fori_loop(
            0, _NCHUNK, chunk_body, init)

        for k in range(_K):
            vm = jnp.maximum(jnp.maximum(v1, v2), jnp.maximum(v3, v4))
            maxv = jnp.max(vm)
            maxv_b = jnp.full((_LANES,), maxv, jnp.float32)
            cand = jnp.minimum(
                jnp.minimum(jnp.where(v1 == maxv_b, i1, big),
                            jnp.where(v2 == maxv_b, i2, big)),
                jnp.minimum(jnp.where(v3 == maxv_b, i3, big),
                            jnp.where(v4 == maxv_b, i4, big)))
            gidx = jnp.min(cand)
            gidx_b = jnp.full((_LANES,), gidx, jnp.int32)
            v1 = jnp.where(i1 == gidx_b, neg1, v1)
            v2 = jnp.where(i2 == gidx_b, neg1, v2)
            v3 = jnp.where(i3 == gidx_b, neg1, v3)
            v4 = jnp.where(i4 == gidx_b, neg1, v4)

            kclamp = jnp.minimum(jnp.maximum(gidx, 1), 127)
            kvec = (jnp.full((_LANES,), kclamp, jnp.int32)
                    + jnp.where(iota == 1, 128, 0))
            vals = plsc.load_gather(lut_v, [kvec])
            off = vals[0].astype(jnp.int32)
            invc_b = jnp.full((_LANES,), vals[1], jnp.float32)
            for jj in range(_PMAX // _LANES):
                seg = fold_v[pl.ds(off + jj * _LANES, _LANES)]
                o_v[pl.ds(k * _PMAX + jj * _LANES, _LANES)] = seg * invc_b

        pltpu.sync_copy(o_v, out_hbm.at[pl.ds(row * (_K * _PMAX),
                                              _K * _PMAX)])
        return _

    lax.fori_loop(0, _ROWS_PER_W, row_body, 0)


@jax.jit
def kernel(x):
    B, T, N = x.shape
    BN = B * N
    seqs = jnp.transpose(x, (0, 2, 1)).reshape(BN, T)
    xh = seqs.astype(jnp.bfloat16)
    xl = (seqs - xh.astype(jnp.float32)).astype(jnp.bfloat16)
    wdft = jnp.asarray(_WDFT)
    wfold = jnp.asarray(_WFOLD)

    rb, cb = 256, 768
    ydft = pl.pallas_call(
        _dft_body,
        grid=(BN // rb, (2 * _FPAD) // cb),
        in_specs=[
            pl.BlockSpec((rb, _T), lambda i, j: (i, 0)),
            pl.BlockSpec((_T, cb), lambda i, j: (0, j)),
        ],
        out_specs=pl.BlockSpec((rb, cb), lambda i, j: (i, j)),
        out_shape=jax.ShapeDtypeStruct((BN, 2 * _FPAD), jnp.float32),
    )(seqs, wdft)

    yfold = pl.pallas_call(
        _fold_body,
        grid=(BN // rb,),
        in_specs=[
            pl.BlockSpec((rb, _T), lambda i: (i, 0)),
            pl.BlockSpec((rb, _T), lambda i: (i, 0)),
            pl.BlockSpec((_T, _FOLDPAD), lambda i: (0, 0)),
        ],
        out_specs=pl.BlockSpec((rb, _FOLDPAD), lambda i: (i, 0)),
        out_shape=jax.ShapeDtypeStruct((BN, _FOLDPAD), jnp.float32),
    )(xh, xl, wfold)

    sc_call = functools.partial(
        pl.kernel,
        mesh=plsc.VectorSubcoreMesh(core_axis_name="c", subcore_axis_name="s"),
        compiler_params=pltpu.CompilerParams(needs_layout_passes=False),
        out_type=jax.ShapeDtypeStruct((BN * _K * _PMAX,), jnp.float32),
        scratch_types=[
            pltpu.VMEM((2 * _FPAD,), jnp.float32),
            pltpu.VMEM((_FOLDPAD,), jnp.float32),
            pltpu.VMEM((_K * _PMAX,), jnp.float32),
            pltpu.VMEM((256,), jnp.float32),
        ],
    )(_sc_select)
    out = sc_call(ydft.reshape(BN * 2 * _FPAD), yfold.reshape(BN * _FOLDPAD),
                  jnp.asarray(_LUT))

    return out.reshape(B, N, _K, _PMAX).transpose(0, 2, 3, 1)
